# baseline (speedup 1.0000x reference)
import functools

import jax
import jax.numpy as jnp
from jax import lax
from jax.experimental import pallas as pl
from jax.experimental.pallas import tpu as pltpu
from jax.experimental.pallas import tpu_sc as plsc


def _make_sc_gather(b):
    info = plsc.get_sparse_core_info()
    nc, ns = info.num_cores, info.num_subcores
    nw = nc * ns
    chunk = 128
    n_chunks = b // chunk // nw

    mesh = plsc.VectorSubcoreMesh(core_axis_name="c", subcore_axis_name="s")

    @functools.partial(
        pl.kernel,
        out_type=jax.ShapeDtypeStruct((b // chunk, chunk, 128), jnp.float32),
        mesh=mesh,
        scratch_types=[
            pltpu.VMEM((n_chunks, chunk), jnp.int32),
            pltpu.VMEM((n_chunks, chunk, 20), jnp.float32),
            pltpu.VMEM((n_chunks, chunk, 128), jnp.float32),
            pltpu.SemaphoreType.DMA,
        ],
        compiler_params=pltpu.CompilerParams(use_tc_tiling_on_sc=False,
                                             needs_layout_passes=False),
    )
    def sc_gather(table_hbm, ids2_hbm, out_hbm, idx_v, rows20_v, rows128_v,
                  sem):
        wid = lax.axis_index("s") * nc + lax.axis_index("c")
        base = wid * n_chunks
        pltpu.sync_copy(ids2_hbm.at[pl.ds(base, n_chunks)], idx_v)
        copies = []
        for j in range(n_chunks):
            copies.append(
                pltpu.async_copy(table_hbm.at[idx_v.at[j]],
                                 rows20_v.at[j], sem))
        for c in copies:
            c.wait()

        lanes = lax.iota(jnp.int32, 16)
        lanes4 = lanes + 4

        def expand_row(r, carry):
            rr = jnp.full((16,), r, jnp.int32)
            for j in range(n_chunks):
                jj = jnp.full((16,), j, jnp.int32)
                x0 = plsc.load_gather(rows20_v, [jj, rr, lanes])
                x1 = plsc.load_gather(rows20_v, [jj, rr, lanes4])
                plsc.store_scatter(rows128_v, [jj, rr, lanes], x0)
                plsc.store_scatter(rows128_v, [jj, rr, lanes4], x1)
            return carry

        lax.fori_loop(0, chunk, expand_row, 0)
        pltpu.sync_copy(rows128_v, out_hbm.at[pl.ds(base, n_chunks)])

    return sc_gather


def _make_softmax_body(k):
    def body(logits_ref, w128_ref, w_ref):
        lg = logits_ref[...]
        lane = lax.broadcasted_iota(jnp.int32, lg.shape, 1)
        lgm = jnp.where(lane < k, lg, jnp.float32(-1e30))
        m = jnp.max(lgm, axis=-1, keepdims=True)
        e = jnp.exp(lgm - m)
        s = jnp.sum(e, axis=-1, keepdims=True)
        w = e / s
        w128_ref[...] = w
        w_ref[...] = w[:, :k]

    return body


def _stream_body(w128_ref, img_ref, fused_ref):
    j = pl.program_id(0)
    onehot = (lax.broadcasted_iota(jnp.int32, (128, 128), 0)
              == j).astype(jnp.float32)
    wcol = jnp.dot(w128_ref[...], onehot,
                   preferred_element_type=jnp.float32)
    contrib = wcol * img_ref[0]

    @pl.when(j == 0)
    def _():
        fused_ref[...] = contrib

    @pl.when(j > 0)
    def _():
        fused_ref[...] += contrib


def kernel(img_views, subject_ids, view_logits_weight):
    b, k, d = img_views.shape

    ids = subject_ids.astype(jnp.int32).reshape(b // 128, 128)
    gather = _make_sc_gather(b)
    logits = gather(view_logits_weight, ids).reshape(b, 128)

    tb_a = 8192
    w128, weights = pl.pallas_call(
        _make_softmax_body(k),
        grid=(b // tb_a,),
        in_specs=[pl.BlockSpec((tb_a, 128), lambda i: (i, 0))],
        out_specs=[
            pl.BlockSpec((tb_a, 128), lambda i: (i, 0)),
            pl.BlockSpec((tb_a, k), lambda i: (i, 0)),
        ],
        out_shape=[
            jax.ShapeDtypeStruct((b, 128), jnp.float32),
            jax.ShapeDtypeStruct((b, k), jnp.float32),
        ],
    )(logits)

    imgT = img_views.transpose(1, 0, 2)

    fused = pl.pallas_call(
        _stream_body,
        grid=(k,),
        in_specs=[
            pl.BlockSpec((b, 128), lambda j: (0, 0)),
            pl.BlockSpec((1, b, d), lambda j: (j, 0, 0)),
        ],
        out_specs=pl.BlockSpec((b, d), lambda j: (0, 0)),
        out_shape=jax.ShapeDtypeStruct((b, d), jnp.float32),
    )(w128, imgT)
    return (fused, weights)
